# baseline (speedup 1.0000x reference)
import jax
import jax.numpy as jnp
from jax.experimental import pallas as pl
from jax.experimental.pallas import tpu as pltpu

f32 = jnp.float32
bf16 = jnp.bfloat16


def _relu(x):
    return jnp.maximum(x, 0.0)


def _ban_main_kernel(v_ref, q_ref,
                     att_v_w_ref, att_v_b_ref, att_q_w_ref, att_q_b_ref,
                     h_mat_ref,
                     bnet_v_w_ref, bnet_v_b_ref, bnet_q_w_ref, bnet_q_b_ref,
                     q_prj_w_ref, q_prj_b_ref,
                     att_ref, x_ref,
                     att_v_wb_ref, att_q_wb_ref,
                     bnet_v_wb_ref, bnet_q_wb_ref, q_prj_wb_ref,
                     xmask_ref):
    BB, V, H = v_ref.shape
    Q = q_ref.shape[1]
    G = bnet_v_w_ref.shape[0]
    BV, BQ = BB * V, BB * Q
    SV, SQ = xmask_ref.shape

    @pl.when(pl.program_id(0) == 0)
    def _init_once():
        att_v_wb_ref[...] = att_v_w_ref[...].astype(bf16)
        att_q_wb_ref[...] = att_q_w_ref[...].astype(bf16)
        bnet_v_wb_ref[...] = bnet_v_w_ref[...].astype(bf16)
        bnet_q_wb_ref[...] = bnet_q_w_ref[...].astype(bf16)
        q_prj_wb_ref[...] = q_prj_w_ref[...].astype(bf16)
        row_b = jax.lax.broadcasted_iota(jnp.int32, (SV, SQ), 0) // V
        col_b = jax.lax.broadcasted_iota(jnp.int32, (SV, SQ), 1) // Q
        xmask_ref[...] = jnp.where(row_b == col_b, 0.0, -jnp.inf).astype(f32)

    att_v_b = att_v_b_ref[...]
    att_q_b = att_q_b_ref[...]
    h_rows = [h_mat_ref[g] for g in range(G)]
    bnet_v_b = [bnet_v_b_ref[g] for g in range(G)]
    bnet_q_b = [bnet_q_b_ref[g] for g in range(G)]
    q_prj_b = [q_prj_b_ref[g] for g in range(G)]

    v2 = v_ref[...].reshape(BV, H)
    q2 = q_ref[...].reshape(BQ, H)
    v2b = v2.astype(bf16)
    q2b = q2.astype(bf16)

    vp3 = _relu(jnp.dot(v2b, att_v_wb_ref[...],
                        preferred_element_type=f32) + att_v_b)
    qp3 = _relu(jnp.dot(q2b, att_q_wb_ref[...],
                        preferred_element_type=f32) + att_q_b)
    qh = qp3.astype(bf16)

    zero_row = jnp.max(jnp.abs(v2b), axis=1, keepdims=True).astype(f32) == 0.0
    maskadd = jnp.where(zero_row, -jnp.inf, 0.0).astype(f32)

    SB = min(BB, SV // V)
    NSB = BB // SB
    SV, SQ = SB * V, SB * Q

    cmasks = [xmask_ref[...] + maskadd[s * SV:(s + 1) * SV] for s in range(NSB)]

    q_cur = q2
    qsum = jnp.sum(q2.reshape(BB, Q, H), axis=1)
    bps = []
    for g in range(G):
        hv = (vp3 * h_rows[g]).astype(bf16)
        p2s = []
        for s in range(NSB):
            lg = jax.lax.dot_general(
                hv[s * SV:(s + 1) * SV], qh[s * SQ:(s + 1) * SQ],
                (((1,), (1,)), ((), ())),
                preferred_element_type=f32)
            lg3 = (lg + cmasks[s]).reshape(SB, V, SQ)
            mx = jnp.max(jnp.max(lg3, axis=2, keepdims=True), axis=1,
                         keepdims=True)
            mx = jnp.where(mx == -jnp.inf, 0.0, mx)
            e = jnp.exp(lg3 - mx)
            su = jnp.sum(jnp.sum(e, axis=2, keepdims=True), axis=1,
                         keepdims=True)
            p3 = e * pl.reciprocal(jnp.maximum(su, 1e-30), approx=True)
            p2 = p3.reshape(SV, SQ)

            folded = p2
            w = SQ
            while w > Q:
                w //= 2
                folded = folded[:, :w] + folded[:, w:]
            att_ref[s * SB:(s + 1) * SB, g, :, :] = folded.reshape(SB, V, Q)
            p2s.append(p2.astype(bf16))

        vp = _relu(jnp.dot(v2b, bnet_v_wb_ref[g],
                           preferred_element_type=f32) + bnet_v_b[g])
        qp = _relu(jnp.dot(q_cur.astype(bf16), bnet_q_wb_ref[g],
                           preferred_element_type=f32) + bnet_q_b[g])
        qpb = qp.astype(bf16)
        bembs = []
        for s in range(NSB):
            tmp = jnp.dot(p2s[s], qpb[s * SQ:(s + 1) * SQ],
                          preferred_element_type=f32)
            prod = tmp * vp[s * SV:(s + 1) * SV]
            bembs.append(jnp.sum(prod.reshape(SB, V, H), axis=1))
        b_emb = bembs[0] if NSB == 1 else jnp.concatenate(bembs, axis=0)
        bp = (jnp.dot(b_emb.astype(bf16), q_prj_wb_ref[g],
                      preferred_element_type=f32) + q_prj_b[g])
        bps.append(bp)
        if g + 1 < G:
            q_cur = (q_cur.reshape(BB, Q, H) + bp[:, None, :]).reshape(BQ, H)

    acc = jnp.zeros_like(bps[0])
    for g in range(G):
        acc = acc + float(G - g) * bps[g]
    x_ref[...] = (float(G) * qsum + float(Q) * acc).astype(bf16)


def _mlp_kernel(x_ref, fc1_w_ref, fc1_b_ref, fc2_w_ref, fc2_b_ref, out_ref,
                h1_ref):
    @pl.when(pl.program_id(0) == 0)
    def _h1_once():
        h1 = _relu(jnp.dot(x_ref[...], fc1_w_ref[...].astype(bf16),
                           preferred_element_type=f32) + fc1_b_ref[...])
        h1_ref[...] = h1.astype(bf16)

    out_ref[...] = (jnp.dot(h1_ref[...], fc2_w_ref[...].astype(bf16),
                            preferred_element_type=f32) + fc2_b_ref[...])


def _ban_forward(v, q, params, *, block_b=16, mlp_block=256, sub_b=16):
    B, V, H = v.shape
    Q = q.shape[1]
    G, _, D3 = params["h_mat"].shape
    H2 = params["fc1_w"].shape[1]
    A = params["fc2_w"].shape[1]
    BB = block_b
    NB = B // BB

    SB = min(BB, sub_b)
    SV, SQ = SB * V, SB * Q

    main_ops = (
        params["att_v_w"], params["att_v_b"].astype(f32),
        params["att_q_w"], params["att_q_b"].astype(f32),
        params["h_mat"].astype(f32),
        params["bnet_v_w"], params["bnet_v_b"].astype(f32),
        params["bnet_q_w"], params["bnet_q_b"].astype(f32),
        params["q_prj_w"], params["q_prj_b"].astype(f32),
    )
    const = lambda shp: pl.BlockSpec(shp, lambda b, _shp=shp: (0,) * len(_shp))
    in_specs = [
        pl.BlockSpec((BB, V, H), lambda b: (b, 0, 0)),
        pl.BlockSpec((BB, Q, H), lambda b: (b, 0, 0)),
        const((H, D3)), const((1, D3)),
        const((H, D3)), const((1, D3)),
        const((G, 1, D3)),
        const((G, H, H)), const((G, 1, H)),
        const((G, H, H)), const((G, 1, H)),
        const((G, H, H)), const((G, 1, H)),
    ]
    out_specs = (
        pl.BlockSpec((BB, G, V, Q), lambda b: (b, 0, 0, 0)),
        pl.BlockSpec((BB, H), lambda b: (b, 0)),
    )
    att, x = pl.pallas_call(
        _ban_main_kernel,
        out_shape=(jax.ShapeDtypeStruct((B, G, V, Q), f32),
                   jax.ShapeDtypeStruct((B, H), bf16)),
        grid=(NB,),
        in_specs=in_specs,
        out_specs=out_specs,
        scratch_shapes=[
            pltpu.VMEM((H, D3), bf16), pltpu.VMEM((H, D3), bf16),
            pltpu.VMEM((G, H, H), bf16), pltpu.VMEM((G, H, H), bf16),
            pltpu.VMEM((G, H, H), bf16),
            pltpu.VMEM((SV, SQ), f32),
        ],
        compiler_params=pltpu.CompilerParams(
            dimension_semantics=("arbitrary",),
            vmem_limit_bytes=100 << 20),
    )(v, q, *main_ops)

    AT = max(1, A // mlp_block)
    AS = A // AT
    out = pl.pallas_call(
        _mlp_kernel,
        out_shape=jax.ShapeDtypeStruct((B, A), f32),
        grid=(AT,),
        in_specs=[
            const((B, H)),
            const((H, H2)), const((1, H2)),
            pl.BlockSpec((H2, AS), lambda n: (0, n)),
            pl.BlockSpec((1, AS), lambda n: (0, n)),
        ],
        out_specs=pl.BlockSpec((B, AS), lambda n: (0, n)),
        scratch_shapes=[pltpu.VMEM((B, H2), bf16)],
        compiler_params=pltpu.CompilerParams(
            dimension_semantics=("arbitrary",),
            vmem_limit_bytes=48 << 20),
    )(x, params["fc1_w"], params["fc1_b"].astype(f32),
      params["fc2_w"], params["fc2_b"].astype(f32))
    return out, att


def kernel(v, q, att_v_w, att_v_b, att_q_w, att_q_b, h_mat, h_bias,
           bnet_v_w, bnet_v_b, bnet_q_w, bnet_q_b, q_prj_w, q_prj_b,
           fc1_w, fc1_b, fc2_w, fc2_b):
    params = {
        "att_v_w": att_v_w, "att_v_b": att_v_b,
        "att_q_w": att_q_w, "att_q_b": att_q_b,
        "h_mat": h_mat,
        "bnet_v_w": bnet_v_w, "bnet_v_b": bnet_v_b,
        "bnet_q_w": bnet_q_w, "bnet_q_b": bnet_q_b,
        "q_prj_w": q_prj_w, "q_prj_b": q_prj_b,
        "fc1_w": fc1_w, "fc1_b": fc1_b,
        "fc2_w": fc2_w, "fc2_b": fc2_b,
    }
    return _ban_forward(v, q, params, block_b=32, mlp_block=768, sub_b=8)


# speedup vs baseline: 1.0073x; 1.0073x over previous
import jax
import jax.numpy as jnp
from jax.experimental import pallas as pl
from jax.experimental.pallas import tpu as pltpu

f32 = jnp.float32
bf16 = jnp.bfloat16


def _relu(x):
    return jnp.maximum(x, 0.0)


def _ban_main_kernel(v_ref, q_ref,
                     att_v_w_ref, att_v_b_ref, att_q_w_ref, att_q_b_ref,
                     h_mat_ref,
                     bnet_v_w_ref, bnet_v_b_ref, bnet_q_w_ref, bnet_q_b_ref,
                     q_prj_w_ref, q_prj_b_ref,
                     att_ref, x_ref,
                     att_v_wb_ref, att_q_wb_ref,
                     bnet_v_wb_ref, bnet_q_wb_ref, q_prj_wb_ref,
                     xmask_ref):
    BB, V, H = v_ref.shape
    Q = q_ref.shape[1]
    G = bnet_v_w_ref.shape[0]
    BV, BQ = BB * V, BB * Q
    SV, SQ = xmask_ref.shape

    @pl.when(pl.program_id(0) == 0)
    def _init_once():
        att_v_wb_ref[...] = att_v_w_ref[...].astype(bf16)
        att_q_wb_ref[...] = att_q_w_ref[...].astype(bf16)
        bnet_v_wb_ref[...] = bnet_v_w_ref[...].astype(bf16)
        bnet_q_wb_ref[...] = bnet_q_w_ref[...].astype(bf16)
        q_prj_wb_ref[...] = q_prj_w_ref[...].astype(bf16)
        row_b = jax.lax.broadcasted_iota(jnp.int32, (SV, SQ), 0) // V
        col_b = jax.lax.broadcasted_iota(jnp.int32, (SV, SQ), 1) // Q
        xmask_ref[...] = jnp.where(row_b == col_b, 0.0, -jnp.inf).astype(f32)

    att_v_b = att_v_b_ref[...]
    att_q_b = att_q_b_ref[...]
    h_rows = [h_mat_ref[g] for g in range(G)]
    bnet_v_b = [bnet_v_b_ref[g] for g in range(G)]
    bnet_q_b = [bnet_q_b_ref[g] for g in range(G)]
    q_prj_b = [q_prj_b_ref[g] for g in range(G)]

    v2 = v_ref[...].reshape(BV, H)
    q2 = q_ref[...].reshape(BQ, H)
    v2b = v2.astype(bf16)
    q2b = q2.astype(bf16)

    vp3 = _relu(jnp.dot(v2b, att_v_wb_ref[...],
                        preferred_element_type=f32) + att_v_b)
    qp3 = _relu(jnp.dot(q2b, att_q_wb_ref[...],
                        preferred_element_type=f32) + att_q_b)
    qh = qp3.astype(bf16)

    zero_row = jnp.max(jnp.abs(v2b), axis=1, keepdims=True).astype(f32) == 0.0
    maskadd = jnp.where(zero_row, -jnp.inf, 0.0).astype(f32)

    SB = min(BB, SV // V)
    NSB = BB // SB
    SV, SQ = SB * V, SB * Q

    cmasks = [xmask_ref[...] + maskadd[s * SV:(s + 1) * SV] for s in range(NSB)]

    q_cur = q2
    qsum = jnp.sum(q2.reshape(BB, Q, H), axis=1)
    bps = []
    for g in range(G):
        hv = (vp3 * h_rows[g]).astype(bf16)
        p2s = []
        for s in range(NSB):
            lg = jax.lax.dot_general(
                hv[s * SV:(s + 1) * SV], qh[s * SQ:(s + 1) * SQ],
                (((1,), (1,)), ((), ())),
                preferred_element_type=f32)
            lg3 = (lg + cmasks[s]).reshape(SB, V, SQ)
            mx = jnp.max(jnp.max(lg3, axis=2, keepdims=True), axis=1,
                         keepdims=True)
            mx = jnp.where(mx == -jnp.inf, 0.0, mx)
            e = jnp.exp(lg3 - mx)
            su = jnp.sum(jnp.sum(e, axis=2, keepdims=True), axis=1,
                         keepdims=True)
            p3 = e * pl.reciprocal(jnp.maximum(su, 1e-30), approx=True)
            p2 = p3.reshape(SV, SQ)

            folded = p2
            w = SQ
            while w > Q:
                w //= 2
                folded = folded[:, :w] + folded[:, w:]
            att_ref[s * SB:(s + 1) * SB, g, :, :] = folded.reshape(SB, V, Q)
            p2s.append(p2.astype(bf16))

        vp = _relu(jnp.dot(v2b, bnet_v_wb_ref[g],
                           preferred_element_type=f32) + bnet_v_b[g])
        qp = _relu(jnp.dot(q_cur.astype(bf16), bnet_q_wb_ref[g],
                           preferred_element_type=f32) + bnet_q_b[g])
        qpb = qp.astype(bf16)
        bembs = []
        for s in range(NSB):
            tmp = jnp.dot(p2s[s], qpb[s * SQ:(s + 1) * SQ],
                          preferred_element_type=f32)
            prod = tmp * vp[s * SV:(s + 1) * SV]
            bembs.append(jnp.sum(prod.reshape(SB, V, H), axis=1))
        b_emb = bembs[0] if NSB == 1 else jnp.concatenate(bembs, axis=0)
        bp = (jnp.dot(b_emb.astype(bf16), q_prj_wb_ref[g],
                      preferred_element_type=f32) + q_prj_b[g])
        bps.append(bp)
        if g + 1 < G:
            q_cur = (q_cur.reshape(BB, Q, H) + bp[:, None, :]).reshape(BQ, H)

    acc = jnp.zeros_like(bps[0])
    for g in range(G):
        acc = acc + float(G - g) * bps[g]
    x_ref[...] = (float(G) * qsum + float(Q) * acc).astype(bf16)


def _mlp_kernel(x_ref, fc1_w_ref, fc1_b_ref, fc2_w_ref, fc2_b_ref, out_ref,
                h1_ref):
    @pl.when(pl.program_id(0) == 0)
    def _h1_once():
        h1 = _relu(jnp.dot(x_ref[...], fc1_w_ref[...].astype(bf16),
                           preferred_element_type=f32) + fc1_b_ref[...])
        h1_ref[...] = h1.astype(bf16)

    out_ref[...] = (jnp.dot(h1_ref[...], fc2_w_ref[...].astype(bf16),
                            preferred_element_type=f32) + fc2_b_ref[...])


def _ban_forward(v, q, params, *, block_b=16, mlp_block=256, sub_b=16):
    B, V, H = v.shape
    Q = q.shape[1]
    G, _, D3 = params["h_mat"].shape
    H2 = params["fc1_w"].shape[1]
    A = params["fc2_w"].shape[1]
    BB = block_b
    NB = B // BB

    SB = min(BB, sub_b)
    SV, SQ = SB * V, SB * Q

    main_ops = (
        params["att_v_w"], params["att_v_b"].astype(f32),
        params["att_q_w"], params["att_q_b"].astype(f32),
        params["h_mat"].astype(f32),
        params["bnet_v_w"], params["bnet_v_b"].astype(f32),
        params["bnet_q_w"], params["bnet_q_b"].astype(f32),
        params["q_prj_w"], params["q_prj_b"].astype(f32),
    )
    const = lambda shp: pl.BlockSpec(shp, lambda b, _shp=shp: (0,) * len(_shp))
    in_specs = [
        pl.BlockSpec((BB, V, H), lambda b: (b, 0, 0)),
        pl.BlockSpec((BB, Q, H), lambda b: (b, 0, 0)),
        const((H, D3)), const((1, D3)),
        const((H, D3)), const((1, D3)),
        const((G, 1, D3)),
        const((G, H, H)), const((G, 1, H)),
        const((G, H, H)), const((G, 1, H)),
        const((G, H, H)), const((G, 1, H)),
    ]
    out_specs = (
        pl.BlockSpec((BB, G, V, Q), lambda b: (b, 0, 0, 0)),
        pl.BlockSpec((BB, H), lambda b: (b, 0)),
    )
    att, x = pl.pallas_call(
        _ban_main_kernel,
        out_shape=(jax.ShapeDtypeStruct((B, G, V, Q), f32),
                   jax.ShapeDtypeStruct((B, H), bf16)),
        grid=(NB,),
        in_specs=in_specs,
        out_specs=out_specs,
        scratch_shapes=[
            pltpu.VMEM((H, D3), bf16), pltpu.VMEM((H, D3), bf16),
            pltpu.VMEM((G, H, H), bf16), pltpu.VMEM((G, H, H), bf16),
            pltpu.VMEM((G, H, H), bf16),
            pltpu.VMEM((SV, SQ), f32),
        ],
        compiler_params=pltpu.CompilerParams(
            dimension_semantics=("arbitrary",),
            vmem_limit_bytes=100 << 20),
    )(v, q, *main_ops)

    AT = max(1, A // mlp_block)
    AS = A // AT
    out = pl.pallas_call(
        _mlp_kernel,
        out_shape=jax.ShapeDtypeStruct((B, A), f32),
        grid=(AT,),
        in_specs=[
            const((B, H)),
            const((H, H2)), const((1, H2)),
            pl.BlockSpec((H2, AS), lambda n: (0, n)),
            pl.BlockSpec((1, AS), lambda n: (0, n)),
        ],
        out_specs=pl.BlockSpec((B, AS), lambda n: (0, n)),
        scratch_shapes=[pltpu.VMEM((B, H2), bf16)],
        compiler_params=pltpu.CompilerParams(
            dimension_semantics=("arbitrary",),
            vmem_limit_bytes=48 << 20),
    )(x, params["fc1_w"], params["fc1_b"].astype(f32),
      params["fc2_w"], params["fc2_b"].astype(f32))
    return out, att


def kernel(v, q, att_v_w, att_v_b, att_q_w, att_q_b, h_mat, h_bias,
           bnet_v_w, bnet_v_b, bnet_q_w, bnet_q_b, q_prj_w, q_prj_b,
           fc1_w, fc1_b, fc2_w, fc2_b):
    params = {
        "att_v_w": att_v_w, "att_v_b": att_v_b,
        "att_q_w": att_q_w, "att_q_b": att_q_b,
        "h_mat": h_mat,
        "bnet_v_w": bnet_v_w, "bnet_v_b": bnet_v_b,
        "bnet_q_w": bnet_q_w, "bnet_q_b": bnet_q_b,
        "q_prj_w": q_prj_w, "q_prj_b": q_prj_b,
        "fc1_w": fc1_w, "fc1_b": fc1_b,
        "fc2_w": fc2_w, "fc2_b": fc2_b,
    }
    return _ban_forward(v, q, params, block_b=32, mlp_block=1536, sub_b=8)
